# baseline (speedup 1.0000x reference)
import functools

import jax
import jax.numpy as jnp
from jax import lax
from jax.experimental import pallas as pl
from jax.experimental.pallas import tpu as pltpu
from jax.experimental.pallas import tpu_sc as plsc

KNB = 8
QT = 256
C = 2048
SUB = C // 8
BIGF = 3.0e38
BIGI = 2**30


def _k1_body(x_ref, xt_ref, xsq_ref, tsq_ref, m_ref):
    ab = jnp.dot(x_ref[...], xt_ref[...], preferred_element_type=jnp.float32)
    d = xsq_ref[...] + tsq_ref[...] - 2.0 * ab
    m = d[:, 0:SUB]
    for j in range(1, 8):
        m = jnp.minimum(m, d[:, j * SUB:(j + 1) * SUB])
    m_ref[...] = m


def _run_k1(x, xt, xsq, tsq, q, npad):
    nqt, nch = q // QT, npad // C
    return pl.pallas_call(
        _k1_body,
        grid=(nqt, nch),
        in_specs=[
            pl.BlockSpec((QT, x.shape[1]), lambda i, c: (i, 0)),
            pl.BlockSpec((x.shape[1], C), lambda i, c: (0, c)),
            pl.BlockSpec((QT, 1), lambda i, c: (i, 0)),
            pl.BlockSpec((1, C), lambda i, c: (0, c)),
        ],
        out_specs=pl.BlockSpec((QT, SUB), lambda i, c: (i, c)),
        out_shape=jax.ShapeDtypeStruct((q, nch * SUB), jnp.float32),
    )(x, xt, xsq, tsq)


def _k2_body(m_ref, ids_ref):
    m = m_ref[...]
    ns = m.shape[1]
    ii = lax.broadcasted_iota(jnp.int32, m.shape, 1)
    cols = []
    for _ in range(KNB):
        mn = jnp.min(m, axis=1, keepdims=True)
        sid = jnp.min(jnp.where(m == mn, ii, BIGI), axis=1, keepdims=True)
        cols.append(sid)
        m = jnp.where(ii == sid, BIGF, m)
    ids_ref[...] = jnp.concatenate(cols, axis=1)


def _run_k2(mins, q):
    ns = mins.shape[1]
    return pl.pallas_call(
        _k2_body,
        grid=(q // QT,),
        in_specs=[pl.BlockSpec((QT, ns), lambda i: (i, 0))],
        out_specs=pl.BlockSpec((QT, KNB), lambda i: (i, 0)),
        out_shape=jax.ShapeDtypeStruct((q, KNB), jnp.int32),
    )(mins)


def _k3_body(x_ref, xt_ref, xsq_ref, tsq_ref, ids_ref, dc_ref):
    c = pl.program_id(1)
    ab = jnp.dot(x_ref[...], xt_ref[...], preferred_element_type=jnp.float32)
    d = xsq_ref[...] + tsq_ref[...] - 2.0 * ab
    ids = ids_ref[...]

    @pl.when(c == 0)
    def _():
        dc_ref[...] = jnp.full(dc_ref.shape, BIGF, jnp.float32)

    si = lax.broadcasted_iota(jnp.int32, (QT, SUB), 1)
    for t in range(KNB):
        cid = ids[:, t]
        in_ch = (cid // SUB) == c
        s = jnp.where(in_ch, cid % SUB, BIGI)
        oh = (si == s[:, None]).astype(jnp.float32)
        vals = [jnp.sum(d[:, j * SUB:(j + 1) * SUB] * oh, axis=1)
                for j in range(8)]
        new = jnp.stack(vals, axis=1)
        prev = dc_ref[:, t * 8:(t + 1) * 8]
        dc_ref[:, t * 8:(t + 1) * 8] = jnp.where(
            in_ch[:, None], new, prev)


def _run_k3(x, xt, xsq, tsq, ids, q, npad):
    nqt, nch = q // QT, npad // C
    return pl.pallas_call(
        _k3_body,
        grid=(nqt, nch),
        in_specs=[
            pl.BlockSpec((QT, x.shape[1]), lambda i, c: (i, 0)),
            pl.BlockSpec((x.shape[1], C), lambda i, c: (0, c)),
            pl.BlockSpec((QT, 1), lambda i, c: (i, 0)),
            pl.BlockSpec((1, C), lambda i, c: (0, c)),
            pl.BlockSpec((QT, KNB), lambda i, c: (i, 0)),
        ],
        out_specs=pl.BlockSpec((QT, KNB * 8), lambda i, c: (i, 0)),
        out_shape=jax.ShapeDtypeStruct((q, KNB * 8), jnp.float32),
    )(x, xt, xsq, tsq, ids)


_GDN = lax.GatherDimensionNumbers(
    offset_dims=(), collapsed_slice_dims=(0,), start_index_map=(0,))


def _perm16(v, perm):
    return lax.gather(v, perm[:, None], _GDN, slice_sizes=(1,),
                      mode=lax.GatherScatterMode.PROMISE_IN_BOUNDS)


def _hmin(v, iota16):
    for sh in (8, 4, 2, 1):
        v = jnp.minimum(v, _perm16(v, iota16 ^ sh))
    return v


def _hmax(v, iota16):
    for sh in (8, 4, 2, 1):
        v = jnp.maximum(v, _perm16(v, iota16 ^ sh))
    return v


def _k4_body(npad, qpw, dc_hbm, ids_hbm, y_hbm, out_hbm,
             dbuf, idsbuf, idxbuf, labbuf, predbuf, sem):
    info = plsc.get_sparse_core_info()
    nc = info.num_cores
    wid = lax.axis_index("s") * nc + lax.axis_index("c")
    base = wid * qpw
    pltpu.sync_copy(dc_hbm.at[pl.ds(base * 64, qpw * 64)], dbuf)
    pltpu.sync_copy(ids_hbm.at[pl.ds(base * 8, qpw * 8)],
                    idsbuf.at[pl.ds(0, qpw * 8)])
    iota16 = lax.iota(jnp.int32, 16)

    def per_pair(pr, carry):
        tvec = jnp.zeros((16,), jnp.int32)
        for half in range(2):
            q = pr * 2 + half
            v = [dbuf[pl.ds(q * 64 + k * 16, 16)] for k in range(4)]
            ids8 = idsbuf[pl.ds(q * 8, 16)]
            for t in range(KNB):
                mn = _hmin(jnp.minimum(jnp.minimum(v[0], v[1]),
                                       jnp.minimum(v[2], v[3])), iota16)
                pc = [jnp.where(v[k] == mn, iota16 + k * 16, jnp.int32(64))
                      for k in range(4)]
                p = _hmin(jnp.minimum(jnp.minimum(pc[0], pc[1]),
                                      jnp.minimum(pc[2], pc[3])), iota16)
                v = [jnp.where(iota16 + k * 16 == p, BIGF, v[k])
                     for k in range(4)]
                slot = p >> 3
                j = p & 7
                cid = _hmax(jnp.where(iota16 == slot, ids8, 0), iota16)
                tidx = (cid >> 8) * C + (cid & (SUB - 1)) + j * SUB
                tvec = jnp.where(iota16 == half * 8 + t, tidx, tvec)
        idxbuf[pl.ds(pr * 16, 16)] = tvec
        return carry

    lax.fori_loop(0, qpw // 2, per_pair, jnp.int32(0))

    for g in range(qpw * 8 // 128):
        pltpu.async_copy(
            y_hbm.at[idxbuf.at[pl.ds(g * 128, 128)]],
            labbuf.at[pl.ds(g * 128, 128)], sem).wait()

    def per_sixteen(i, carry):
        pvec = jnp.zeros((16,), jnp.int32)
        for u in range(8):
            labs = labbuf[pl.ds((i * 8 + u) * 16, 16)]
            half = iota16 & 7
            hbase = iota16 & 8
            cnt = jnp.full((16,), 1, jnp.int32)
            for j in range(1, 8):
                perm = ((half + j) & 7) | hbase
                rolled = _perm16(labs, perm)
                cnt = cnt + jnp.where(rolled == labs, 1, 0)
            score = cnt * 1024 + (1023 - labs)
            s0 = _hmax(jnp.where(iota16 < 8, score, 0), iota16)
            s1 = _hmax(jnp.where(iota16 >= 8, score, 0), iota16)
            pvec = jnp.where(iota16 == 2 * u, 1023 - (s0 & 1023), pvec)
            pvec = jnp.where(iota16 == 2 * u + 1,
                             1023 - (s1 & 1023), pvec)
        predbuf[pl.ds(i * 16, 16)] = pvec
        return carry

    lax.fori_loop(0, qpw // 16, per_sixteen, jnp.int32(0))
    pltpu.sync_copy(predbuf, out_hbm.at[pl.ds(base, qpw)])


def _run_k4(dc_flat, ids_flat, y_pad, q, npad):
    info = plsc.get_sparse_core_info()
    nw = info.num_cores * info.num_subcores
    qpw = q // nw
    mesh = plsc.VectorSubcoreMesh(core_axis_name="c", subcore_axis_name="s")
    kern = functools.partial(
        pl.kernel,
        mesh=mesh,
        out_type=jax.ShapeDtypeStruct((q,), jnp.int32),
        scratch_types=[
            pltpu.VMEM((qpw * 64,), jnp.float32),
            pltpu.VMEM((qpw * 8 + 16,), jnp.int32),
            pltpu.VMEM((qpw * 8,), jnp.int32),
            pltpu.VMEM((qpw * 8,), jnp.int32),
            pltpu.VMEM((qpw,), jnp.int32),
            pltpu.SemaphoreType.DMA,
        ],
    )(functools.partial(_k4_body, npad, qpw))
    return kern(dc_flat, ids_flat, y_pad)


def kernel(x, x_train, y_train):
    q, dim = x.shape
    n = x_train.shape[0]
    npad = ((n + C - 1) // C) * C

    xsq = jnp.sum(x * x, axis=1, keepdims=True)
    tsq = jnp.sum(x_train * x_train, axis=1)
    tsq = jnp.pad(tsq, (0, npad - n), constant_values=1e30)[None, :]
    xt = jnp.pad(x_train, ((0, npad - n), (0, 0))).T
    xt = jnp.asarray(xt, jnp.float32)
    y_pad = jnp.pad(y_train, (0, npad - n))

    mins = _run_k1(x, xt, xsq, tsq, q, npad)
    ids = _run_k2(mins, q)
    dc = _run_k3(x, xt, xsq, tsq, ids, q, npad)
    preds = _run_k4(dc.reshape(-1), ids.reshape(-1), y_pad, q, npad)
    return preds


# speedup vs baseline: 1.9589x; 1.9589x over previous
import functools

import jax
import jax.numpy as jnp
from jax import lax
from jax.experimental import pallas as pl
from jax.experimental.pallas import tpu as pltpu
from jax.experimental.pallas import tpu_sc as plsc

KNB = 8
QT = 256
C = 2048
SUB = C // 8
BIGF = 3.0e38
BIGI = 2**30


def _k1_body(x_ref, xt_ref, xsq_ref, tsq_ref, m_ref, d_ref):
    ab = jnp.dot(x_ref[...], xt_ref[...], preferred_element_type=jnp.float32)
    d = xsq_ref[...] + tsq_ref[...] - 2.0 * ab
    d_ref[...] = d
    m = d[:, 0:SUB]
    for j in range(1, 8):
        m = jnp.minimum(m, d[:, j * SUB:(j + 1) * SUB])
    m_ref[...] = m


def _run_k1(x, xt, xsq, tsq, q, npad):
    nqt, nch = q // QT, npad // C
    return pl.pallas_call(
        _k1_body,
        grid=(nqt, nch),
        in_specs=[
            pl.BlockSpec((QT, x.shape[1]), lambda i, c: (i, 0)),
            pl.BlockSpec((x.shape[1], C), lambda i, c: (0, c)),
            pl.BlockSpec((QT, 1), lambda i, c: (i, 0)),
            pl.BlockSpec((1, C), lambda i, c: (0, c)),
        ],
        out_specs=[
            pl.BlockSpec((QT, SUB), lambda i, c: (i, c)),
            pl.BlockSpec((QT, C), lambda i, c: (i, c)),
        ],
        out_shape=[
            jax.ShapeDtypeStruct((q, nch * SUB), jnp.float32),
            jax.ShapeDtypeStruct((q, npad), jnp.float32),
        ],
    )(x, xt, xsq, tsq)


def _k2_body(m_ref, ids_ref):
    m = m_ref[...]
    ns = m.shape[1]
    ii = lax.broadcasted_iota(jnp.int32, m.shape, 1)
    cols = []
    for _ in range(KNB):
        mn = jnp.min(m, axis=1, keepdims=True)
        sid = jnp.min(jnp.where(m == mn, ii, BIGI), axis=1, keepdims=True)
        cols.append(sid)
        m = jnp.where(ii == sid, BIGF, m)
    ids_ref[...] = jnp.concatenate(cols, axis=1)


def _run_k2(mins, q):
    ns = mins.shape[1]
    return pl.pallas_call(
        _k2_body,
        grid=(q // QT,),
        in_specs=[pl.BlockSpec((QT, ns), lambda i: (i, 0))],
        out_specs=pl.BlockSpec((QT, KNB), lambda i: (i, 0)),
        out_shape=jax.ShapeDtypeStruct((q, KNB), jnp.int32),
    )(mins)


_GDN = lax.GatherDimensionNumbers(
    offset_dims=(), collapsed_slice_dims=(0,), start_index_map=(0,))


def _perm16(v, perm):
    return lax.gather(v, perm[:, None], _GDN, slice_sizes=(1,),
                      mode=lax.GatherScatterMode.PROMISE_IN_BOUNDS)


def _hmin(v, iota16):
    for sh in (8, 4, 2, 1):
        v = jnp.minimum(v, _perm16(v, iota16 ^ sh))
    return v


def _hmax(v, iota16):
    for sh in (8, 4, 2, 1):
        v = jnp.maximum(v, _perm16(v, iota16 ^ sh))
    return v


def _k4_body(npad, qpw, df_hbm, ids_hbm, y_hbm, out_hbm,
             dbuf, gibuf, idsbuf, idxbuf, labbuf, predbuf, sem):
    info = plsc.get_sparse_core_info()
    nc = info.num_cores
    wid = lax.axis_index("s") * nc + lax.axis_index("c")
    base = wid * qpw
    pltpu.sync_copy(ids_hbm.at[pl.ds(base * 8, qpw * 8)],
                    idsbuf.at[pl.ds(0, qpw * 8)])
    iota16 = lax.iota(jnp.int32, 16)

    def build_idx(pr, carry):
        ids16 = idsbuf[pl.ds(pr * 16, 16)]
        for k in range(8):
            second = 1 if k >= 4 else 0
            qq = base + pr * 2 + second
            cand = (k % 4) * 16 + iota16
            t = cand >> 3
            j = cand & 7
            cidv = _perm16(ids16, t + 8 * second)
            gidx = (qq * npad + (cidv >> 8) * C
                    + (cidv & (SUB - 1)) + j * SUB)
            gibuf[pl.ds(pr * 128 + k * 16, 16)] = gidx
        return carry

    lax.fori_loop(0, qpw // 2, build_idx, jnp.int32(0))

    copies = [
        pltpu.async_copy(
            df_hbm.at[gibuf.at[pl.ds(g * 128, 128)]],
            dbuf.at[pl.ds(g * 128, 128)], sem)
        for g in range(qpw * 64 // 128)
    ]
    for cp in copies:
        cp.wait()

    def per_pair(pr, carry):
        tvec = jnp.zeros((16,), jnp.int32)
        for half in range(2):
            q = pr * 2 + half
            v = [dbuf[pl.ds(q * 64 + k * 16, 16)] for k in range(4)]
            ids8 = idsbuf[pl.ds(q * 8, 16)]
            for t in range(KNB):
                mn = _hmin(jnp.minimum(jnp.minimum(v[0], v[1]),
                                       jnp.minimum(v[2], v[3])), iota16)
                pc = [jnp.where(v[k] == mn, iota16 + k * 16, jnp.int32(64))
                      for k in range(4)]
                p = _hmin(jnp.minimum(jnp.minimum(pc[0], pc[1]),
                                      jnp.minimum(pc[2], pc[3])), iota16)
                v = [jnp.where(iota16 + k * 16 == p, BIGF, v[k])
                     for k in range(4)]
                slot = p >> 3
                j = p & 7
                cid = _hmax(jnp.where(iota16 == slot, ids8, 0), iota16)
                tidx = (cid >> 8) * C + (cid & (SUB - 1)) + j * SUB
                tvec = jnp.where(iota16 == half * 8 + t, tidx, tvec)
        idxbuf[pl.ds(pr * 16, 16)] = tvec
        return carry

    lax.fori_loop(0, qpw // 2, per_pair, jnp.int32(0))

    for g in range(qpw * 8 // 128):
        pltpu.async_copy(
            y_hbm.at[idxbuf.at[pl.ds(g * 128, 128)]],
            labbuf.at[pl.ds(g * 128, 128)], sem).wait()

    def per_sixteen(i, carry):
        pvec = jnp.zeros((16,), jnp.int32)
        for u in range(8):
            labs = labbuf[pl.ds((i * 8 + u) * 16, 16)]
            half = iota16 & 7
            hbase = iota16 & 8
            cnt = jnp.full((16,), 1, jnp.int32)
            for j in range(1, 8):
                perm = ((half + j) & 7) | hbase
                rolled = _perm16(labs, perm)
                cnt = cnt + jnp.where(rolled == labs, 1, 0)
            score = cnt * 1024 + (1023 - labs)
            s0 = _hmax(jnp.where(iota16 < 8, score, 0), iota16)
            s1 = _hmax(jnp.where(iota16 >= 8, score, 0), iota16)
            pvec = jnp.where(iota16 == 2 * u, 1023 - (s0 & 1023), pvec)
            pvec = jnp.where(iota16 == 2 * u + 1,
                             1023 - (s1 & 1023), pvec)
        predbuf[pl.ds(i * 16, 16)] = pvec
        return carry

    lax.fori_loop(0, qpw // 16, per_sixteen, jnp.int32(0))
    pltpu.sync_copy(predbuf, out_hbm.at[pl.ds(base, qpw)])


def _run_k4(df_flat, ids_flat, y_pad, q, npad):
    info = plsc.get_sparse_core_info()
    nw = info.num_cores * info.num_subcores
    qpw = q // nw
    mesh = plsc.VectorSubcoreMesh(core_axis_name="c", subcore_axis_name="s")
    kern = functools.partial(
        pl.kernel,
        mesh=mesh,
        out_type=jax.ShapeDtypeStruct((q,), jnp.int32),
        scratch_types=[
            pltpu.VMEM((qpw * 64,), jnp.float32),
            pltpu.VMEM((qpw * 64,), jnp.int32),
            pltpu.VMEM((qpw * 8 + 16,), jnp.int32),
            pltpu.VMEM((qpw * 8,), jnp.int32),
            pltpu.VMEM((qpw * 8,), jnp.int32),
            pltpu.VMEM((qpw,), jnp.int32),
            pltpu.SemaphoreType.DMA,
        ],
    )(functools.partial(_k4_body, npad, qpw))
    return kern(df_flat, ids_flat, y_pad)


def kernel(x, x_train, y_train):
    q, dim = x.shape
    n = x_train.shape[0]
    npad = ((n + C - 1) // C) * C

    xsq = jnp.sum(x * x, axis=1, keepdims=True)
    tsq = jnp.sum(x_train * x_train, axis=1)
    tsq = jnp.pad(tsq, (0, npad - n), constant_values=1e30)[None, :]
    xt = jnp.pad(x_train, ((0, npad - n), (0, 0))).T
    xt = jnp.asarray(xt, jnp.float32)
    y_pad = jnp.pad(y_train, (0, npad - n))

    mins, dfull = _run_k1(x, xt, xsq, tsq, q, npad)
    ids = _run_k2(mins, q)
    preds = _run_k4(dfull.reshape(-1), ids.reshape(-1), y_pad, q, npad)
    return preds
